# Initial kernel scaffold; baseline (speedup 1.0000x reference)
import jax
import jax.numpy as jnp
from jax.experimental import pallas as pl


def kernel(x_user, x_item, edge_u2i, edge_i2u, lin_w_user, lin_b_user, lin_w_item, lin_b_item, w0_u2i, b0_u2i, w0_i2u, b0_i2u, w1_u2i, b1_u2i, w1_i2u, b1_i2u, w2_u2i, b2_u2i, w2_i2u, b2_i2u, g0_user, be0_user, g0_item, be0_item, g1_user, be1_user, g1_item, be1_item):
    raise NotImplementedError("write your pallas kernel here")


# speedup vs baseline: 2.2182x; 2.2182x over previous
import jax
import jax.numpy as jnp
from jax import lax
from jax.experimental import pallas as pl
from jax.experimental.pallas import tpu as pltpu
from jax.experimental.pallas import tpu_sc as plsc

N_U = 100000
N_I = 10000
E0 = 500000
D = 128
EPS = 1e-5

EPAD = 524288
NROWS = EPAD // 128
RPW = NROWS // 16
J = 4
NBLK = RPW // J
NUP = 100352
NIP = 10240
NCHUNK = 8


def _run_dir(table, srcr, dstr, out, ndst, sid, acc, idx_s, idx_d, gath,
             zeros_h, gsem):
  rows_w = ndst // 16
  base = sid * rows_w
  for c in range(NCHUNK):
    tbl_c = table.at[c]
    pltpu.sync_copy(zeros_h.at[pl.ds(0, rows_w)],
                    acc.at[pl.ds(base, rows_w)])
    plsc.subcore_barrier()

    def blk(b, carry):
      row0 = sid * RPW + b * J
      pltpu.sync_copy(srcr.at[pl.ds(row0, J)], idx_s)
      pltpu.sync_copy(dstr.at[pl.ds(row0, J)], idx_d)
      pltpu.async_copy(tbl_c.at[idx_s.at[0]], gath.at[pl.ds(0, 128)],
                       gsem)

      def jb(j, c2):
        pltpu.make_async_copy(
            tbl_c.at[idx_s.at[j]],
            gath.at[pl.ds(j * 128, 128)], gsem).wait()

        @pl.when(j < J - 1)
        def _():
          pltpu.async_copy(tbl_c.at[idx_s.at[j + 1]],
                           gath.at[pl.ds((j + 1) * 128, 128)], gsem)

        pltpu.sync_copy(gath.at[pl.ds(j * 128, 128)], acc.at[idx_d.at[j]],
                        add=True)
        return c2

      lax.fori_loop(0, J, jb, 0)
      return carry

    lax.fori_loop(0, NBLK, blk, 0)
    plsc.subcore_barrier()
    pltpu.sync_copy(acc.at[pl.ds(base, rows_w)],
                    out.at[c].at[pl.ds(base, rows_w)])


def _sc_agg_body(zeros_h, tu, ti, su_r, di_r, si_r, du_r, agg_u, agg_i,
                 acc, idx_s, idx_d, gath, gsem):
  cid = lax.axis_index("c")
  sid = lax.axis_index("s")

  @pl.when(cid == 0)
  def _():
    _run_dir(ti, si_r, du_r, agg_u, NUP, sid, acc, idx_s, idx_d, gath,
             zeros_h, gsem)

  @pl.when(cid == 1)
  def _():
    _run_dir(tu, su_r, di_r, agg_i, NIP, sid, acc, idx_s, idx_d, gath,
             zeros_h, gsem)


def _sc_agg(zeros_h, tu, ti, su_r, di_r, si_r, du_r):
  mesh = plsc.VectorSubcoreMesh(core_axis_name="c", subcore_axis_name="s")
  return pl.kernel(
      _sc_agg_body,
      out_type=(jax.ShapeDtypeStruct((NCHUNK, NUP, 16), jnp.float32),
                jax.ShapeDtypeStruct((NCHUNK, NIP, 16), jnp.float32)),
      mesh=mesh,
      scratch_types=[
          pltpu.VMEM_SHARED((NUP, 16), jnp.float32),
          pltpu.VMEM((J, 128), jnp.int32),
          pltpu.VMEM((J, 128), jnp.int32),
          pltpu.VMEM((J * 128, 16), jnp.float32),
          pltpu.SemaphoreType.DMA,
      ],
      compiler_params=pltpu.CompilerParams(use_tc_tiling_on_sc=False),
  )(zeros_h, tu, ti, su_r, di_r, si_r, du_r)


def _hist(dstr, out, ndst, sid, acc, idx_d, onesv, zeros_h):
  rows_w = ndst // 16
  base = sid * rows_w
  pltpu.sync_copy(zeros_h.at[pl.ds(0, rows_w)], acc.at[pl.ds(base, rows_w)])
  plsc.subcore_barrier()

  def blk(b, carry):
    row0 = sid * RPW + b * J
    pltpu.sync_copy(dstr.at[pl.ds(row0, J)], idx_d)

    def jb(j, c2):
      pltpu.sync_copy(onesv, acc.at[idx_d.at[j]], add=True)
      return c2

    lax.fori_loop(0, J, jb, 0)
    return carry

  lax.fori_loop(0, NBLK, blk, 0)
  plsc.subcore_barrier()
  pltpu.sync_copy(acc.at[pl.ds(base, rows_w)],
                  out.at[pl.ds(base, rows_w)])


def _sc_deg_body(zeros_h, ones_h, du_r, si_r, su_r, di_r,
                 deg_du, deg_si, deg_su, deg_di,
                 acc, idx_d, onesv):
  cid = lax.axis_index("c")
  sid = lax.axis_index("s")
  pltpu.sync_copy(ones_h, onesv)

  @pl.when(cid == 0)
  def _():
    _hist(du_r, deg_du, NUP, sid, acc, idx_d, onesv, zeros_h)
    _hist(si_r, deg_si, NIP, sid, acc, idx_d, onesv, zeros_h)

  @pl.when(cid == 1)
  def _():
    _hist(su_r, deg_su, NUP, sid, acc, idx_d, onesv, zeros_h)
    _hist(di_r, deg_di, NIP, sid, acc, idx_d, onesv, zeros_h)


def _sc_degrees(zeros_h, ones_h, du_r, si_r, su_r, di_r):
  mesh = plsc.VectorSubcoreMesh(core_axis_name="c", subcore_axis_name="s")
  return pl.kernel(
      _sc_deg_body,
      out_type=(jax.ShapeDtypeStruct((NUP, 16), jnp.float32),
                jax.ShapeDtypeStruct((NIP, 16), jnp.float32),
                jax.ShapeDtypeStruct((NUP, 16), jnp.float32),
                jax.ShapeDtypeStruct((NIP, 16), jnp.float32)),
      mesh=mesh,
      scratch_types=[
          pltpu.VMEM_SHARED((NUP, 16), jnp.float32),
          pltpu.VMEM((J, 128), jnp.int32),
          pltpu.VMEM((128, 16), jnp.float32),
      ],
      compiler_params=pltpu.CompilerParams(use_tc_tiling_on_sc=False),
  )(zeros_h, ones_h, du_r, si_r, su_r, di_r)


def _dis(deg_col):
  d = deg_col
  return jnp.where(d > 0, lax.rsqrt(jnp.maximum(d, 1.0)), 0.0)


def _row_mask(bu, nvalid):
  rid = pl.program_id(0) * bu + lax.broadcasted_iota(jnp.int32, (bu, 1), 0)
  return rid < nvalid


def _cat(agg):
  return jnp.concatenate([agg[j] for j in range(NCHUNK)], axis=-1)


def _split_store(o, val):
  for j in range(NCHUNK):
    o[j] = val[:, j * 16:(j + 1) * 16]


def _user_table0_body(x, w, b, deg, o):
  bu = o.shape[1]
  val = (jnp.dot(x[...], w[...], preferred_element_type=jnp.float32)
         + b[...]) * _dis(deg[:, 0:1])
  _split_store(o, jnp.where(_row_mask(bu, N_U), val, 0.0))


def _user_mid_body(agg, degd, degs, s, c2, o):
  bu = o.shape[1]
  h = jnp.maximum(_cat(agg) * _dis(degd[:, 0:1]) * s[...] + c2[...], 0.0)
  _split_store(o, jnp.where(_row_mask(bu, N_U), h * _dis(degs[:, 0:1]), 0.0))


def _user_fin_body(agg, degd, b, o):
  o[...] = _cat(agg) * _dis(degd[:, 0:1]) + b[...]


def _item_table0_body(x, w1, b1, w2, deg, o):
  bu = o.shape[1]
  h = jnp.dot(x[...], w1[...], preferred_element_type=jnp.float32) + b1[...]
  val = jnp.dot(h, w2[...], preferred_element_type=jnp.float32) \
      * _dis(deg[:, 0:1])
  _split_store(o, jnp.where(_row_mask(bu, N_I), val, 0.0))


def _item_mid_body(agg, degd, degs, w1, s, c2, w2, o):
  bu = o.shape[1]
  oi = jnp.dot(_cat(agg) * _dis(degd[:, 0:1]), w1[...],
               preferred_element_type=jnp.float32)
  h = jnp.maximum(oi * s[...] + c2[...], 0.0)
  val = jnp.dot(h, w2[...], preferred_element_type=jnp.float32) \
      * _dis(degs[:, 0:1])
  _split_store(o, jnp.where(_row_mask(bu, N_I), val, 0.0))


def _item_fin_body(agg, degd, w, b, o):
  oi = jnp.dot(_cat(agg) * _dis(degd[:, 0:1]), w[...],
               preferred_element_type=jnp.float32)
  o[...] = oi + b[...]


_BU = 512


def _rows_spec(width):
  return pl.BlockSpec((_BU, width), lambda i: (i, 0))


def _chunk_spec():
  return pl.BlockSpec((NCHUNK, _BU, 16), lambda i: (0, i, 0))


def _full_spec(shape):
  return pl.BlockSpec(shape, lambda i: tuple(0 for _ in shape))


def _tc_call(body, nrows, in_specs, args, chunked_out=True):
  if chunked_out:
    out_specs = _chunk_spec()
    out_shape = jax.ShapeDtypeStruct((NCHUNK, nrows, 16), jnp.float32)
  else:
    out_specs = _rows_spec(D)
    out_shape = jax.ShapeDtypeStruct((nrows, D), jnp.float32)
  return pl.pallas_call(
      body,
      grid=(nrows // _BU,),
      in_specs=in_specs,
      out_specs=out_specs,
      out_shape=out_shape,
  )(*args)


def _user_table0(x, w, b, deg):
  return _tc_call(
      _user_table0_body, NUP,
      [_rows_spec(D), _full_spec((D, D)), _full_spec((1, D)),
       _rows_spec(16)],
      (x, w, b.reshape(1, D), deg))


def _user_mid(agg, degd, degs, s, c2):
  return _tc_call(
      _user_mid_body, NUP,
      [_chunk_spec(), _rows_spec(16), _rows_spec(16), _full_spec((1, D)),
       _full_spec((1, D))],
      (agg, degd, degs, s.reshape(1, D), c2.reshape(1, D)))


def _user_fin(agg, degd, b):
  return _tc_call(
      _user_fin_body, NUP,
      [_chunk_spec(), _rows_spec(16), _full_spec((1, D))],
      (agg, degd, b.reshape(1, D)), chunked_out=False)


def _item_table0(x, w1, b1, w2, deg):
  return _tc_call(
      _item_table0_body, NIP,
      [_rows_spec(D), _full_spec((D, D)), _full_spec((1, D)),
       _full_spec((D, D)), _rows_spec(16)],
      (x, w1, b1.reshape(1, D), w2, deg))


def _item_mid(agg, degd, degs, w1, s, c2, w2):
  return _tc_call(
      _item_mid_body, NIP,
      [_chunk_spec(), _rows_spec(16), _rows_spec(16), _full_spec((D, D)),
       _full_spec((1, D)), _full_spec((1, D)), _full_spec((D, D))],
      (agg, degd, degs, w1, s.reshape(1, D), c2.reshape(1, D), w2))


def _item_fin(agg, degd, w, b):
  return _tc_call(
      _item_fin_body, NIP,
      [_chunk_spec(), _rows_spec(16), _full_spec((D, D)), _full_spec((1, D))],
      (agg, degd, w, b.reshape(1, D)), chunked_out=False)


def _pad_idx(idx, fill):
  p = jnp.concatenate(
      [idx, jnp.full((EPAD - E0,), fill, jnp.int32)])
  return p.reshape(NROWS, 128)


def kernel(x_user, x_item, edge_u2i, edge_i2u, lin_w_user, lin_b_user,
           lin_w_item, lin_b_item, w0_u2i, b0_u2i, w0_i2u, b0_i2u,
           w1_u2i, b1_u2i, w1_i2u, b1_i2u, w2_u2i, b2_u2i, w2_i2u, b2_i2u,
           g0_user, be0_user, g0_item, be0_item, g1_user, be1_user,
           g1_item, be1_item):
  su_r = _pad_idx(edge_u2i[0], N_U)
  di_r = _pad_idx(edge_u2i[1], N_I)
  si_r = _pad_idx(edge_i2u[0], N_I)
  du_r = _pad_idx(edge_i2u[1], N_U)

  zeros_h = jnp.zeros((NUP // 16, 16), jnp.float32)
  ones_h = jnp.ones((128, 16), jnp.float32)

  deg_du, deg_si, deg_su, deg_di = _sc_degrees(
      zeros_h, ones_h, du_r, si_r, su_r, di_r)

  inv = 1.0 / jnp.sqrt(jnp.float32(1.0 + EPS))
  s_u = (g0_user * inv, g1_user * inv)
  c2_u = (b0_i2u * s_u[0] + be0_user, b1_i2u * s_u[1] + be1_user)
  s_i = (g0_item * inv, g1_item * inv)
  c2_i = (b0_u2i * s_i[0] + be0_item, b1_u2i * s_i[1] + be1_item)
  w_u2i = (w0_u2i, w1_u2i, w2_u2i)
  w_i2u_next = (w1_i2u, w2_i2u)

  tu = _user_table0(x_user, lin_w_user, lin_b_user, deg_su)
  ti = _item_table0(x_item, lin_w_item, lin_b_item, w0_i2u, deg_si)

  for l in range(2):
    agg_u, agg_i = _sc_agg(zeros_h, tu, ti, su_r, di_r, si_r, du_r)
    tu = _user_mid(agg_u, deg_du, deg_su, s_u[l], c2_u[l])
    ti = _item_mid(agg_i, deg_di, deg_si, w_u2i[l], s_i[l], c2_i[l],
                   w_i2u_next[l])

  agg_u, agg_i = _sc_agg(zeros_h, tu, ti, su_r, di_r, si_r, du_r)
  out_u = _user_fin(agg_u, deg_du, b2_i2u)[:N_U]
  out_i = _item_fin(agg_i, deg_di, w2_u2i, b2_u2i)[:N_I]
  return (out_u, out_i)


# speedup vs baseline: 2.5023x; 1.1281x over previous
import jax
import jax.numpy as jnp
from jax import lax
from jax.experimental import pallas as pl
from jax.experimental.pallas import tpu as pltpu
from jax.experimental.pallas import tpu_sc as plsc

N_U = 100000
N_I = 10000
E0 = 500000
D = 128
EPS = 1e-5

EPAD = 524288
NROWS = EPAD // 128
RPW = NROWS // 16
J = 4
NBLK = RPW // J
ZRU = 196
ZRI = 160
NSB = NBLK // 2
NUP = 100352
NIP = 10240
NCHUNK = 8


def _run_dir(table, srcr, dstr, out, ndst, sid, acc, ibuf_s, ibuf_d,
             gath, zeros_h, gsems, ssem):
  rows_w = ndst // 16
  base = sid * rows_w
  zr = ZRU if ndst == NUP else ZRI
  nzp = rows_w // zr
  for c in range(NCHUNK):
    tbl_c = table.at[c]

    def zfill(z, cr):
      pltpu.sync_copy(zeros_h.at[pl.ds(0, zr)],
                      acc.at[pl.ds(base + z * zr, zr)])
      return cr

    lax.fori_loop(0, nzp, zfill, 0)
    plsc.subcore_barrier()

    def load_idx(b, ph):
      row0 = sid * RPW + b * J
      pltpu.sync_copy(srcr.at[pl.ds(row0, J)], ibuf_s[ph])
      pltpu.sync_copy(dstr.at[pl.ds(row0, J)], ibuf_d[ph])

    def fire_gathers(ph):
      def f(j, cr):
        pltpu.async_copy(tbl_c.at[ibuf_s[ph].at[j]],
                         gath[ph].at[pl.ds(j * 128, 128)], gsems[ph])
        return cr
      lax.fori_loop(0, J, f, 0)

    def wait_gathers(ph):
      def f(j, cr):
        pltpu.make_async_copy(tbl_c.at[ibuf_s[ph].at[j]],
                              gath[ph].at[pl.ds(j * 128, 128)],
                              gsems[ph]).wait()
        return cr
      lax.fori_loop(0, J, f, 0)

    def fire_scatters(ph):
      def f(j, cr):
        pltpu.async_copy(gath[ph].at[pl.ds(j * 128, 128)],
                         acc.at[ibuf_d[ph].at[j]], ssem, add=True)
        return cr
      lax.fori_loop(0, J, f, 0)

    def drain_scatters(ph):
      def f(j, cr):
        pltpu.make_async_copy(zeros_h.at[pl.ds(0, 128)],
                              gath[ph].at[pl.ds(j * 128, 128)], ssem).wait()
        return cr
      lax.fori_loop(0, J, f, 0)

    load_idx(0, 0)
    fire_gathers(0)

    def sb_body(sb, carry):
      for ph in (0, 1):
        b = sb * 2 + ph
        if ph == 1:
          drain_scatters(0)
        else:
          @pl.when(sb > 0)
          def _():
            drain_scatters(1)
        wait_gathers(ph)
        if ph == 0:
          load_idx(b + 1, 1)
          fire_gathers(1)
        else:
          @pl.when(sb < NSB - 1)
          def _():
            load_idx(b + 1, 0)
            fire_gathers(0)
        fire_scatters(ph)
      return carry

    lax.fori_loop(0, NSB, sb_body, 0)
    drain_scatters(1)
    plsc.subcore_barrier()
    out_c = out.at[c]

    def wout(z, cr):
      pltpu.sync_copy(acc.at[pl.ds(base + z * zr, zr)],
                      out_c.at[pl.ds(base + z * zr, zr)])
      return cr

    lax.fori_loop(0, nzp, wout, 0)


def _sc_agg_body(zeros_h, tu, ti, su_r, di_r, si_r, du_r, agg_u, agg_i,
                 acc, is0, is1, id0, id1, g0, g1, gsem0, gsem1, ssem):
  cid = lax.axis_index("c")
  sid = lax.axis_index("s")
  ibuf_s = (is0, is1)
  ibuf_d = (id0, id1)
  gath = (g0, g1)
  gsems = (gsem0, gsem1)

  @pl.when(cid == 0)
  def _():
    _run_dir(ti, si_r, du_r, agg_u, NUP, sid, acc, ibuf_s, ibuf_d, gath,
             zeros_h, gsems, ssem)

  @pl.when(cid == 1)
  def _():
    _run_dir(tu, su_r, di_r, agg_i, NIP, sid, acc, ibuf_s, ibuf_d, gath,
             zeros_h, gsems, ssem)


def _sc_agg(zeros_h, tu, ti, su_r, di_r, si_r, du_r):
  mesh = plsc.VectorSubcoreMesh(core_axis_name="c", subcore_axis_name="s")
  return pl.kernel(
      _sc_agg_body,
      out_type=(jax.ShapeDtypeStruct((NCHUNK, NUP, 16), jnp.float32),
                jax.ShapeDtypeStruct((NCHUNK, NIP, 16), jnp.float32)),
      mesh=mesh,
      scratch_types=[
          pltpu.VMEM_SHARED((NUP, 16), jnp.float32),
          pltpu.VMEM((J, 128), jnp.int32),
          pltpu.VMEM((J, 128), jnp.int32),
          pltpu.VMEM((J, 128), jnp.int32),
          pltpu.VMEM((J, 128), jnp.int32),
          pltpu.VMEM((J * 128, 16), jnp.float32),
          pltpu.VMEM((J * 128, 16), jnp.float32),
          pltpu.SemaphoreType.DMA,
          pltpu.SemaphoreType.DMA,
          pltpu.SemaphoreType.DMA,
      ],
      compiler_params=pltpu.CompilerParams(use_tc_tiling_on_sc=False),
  )(zeros_h, tu, ti, su_r, di_r, si_r, du_r)


def _hist(dstr, out, ndst, sid, acc, idx_d, onesv, zeros_h):
  rows_w = ndst // 16
  base = sid * rows_w
  zr = ZRU if ndst == NUP else ZRI
  nzp = rows_w // zr

  def zfill(z, cr):
    pltpu.sync_copy(zeros_h.at[pl.ds(0, zr)],
                    acc.at[pl.ds(base + z * zr, zr)])
    return cr

  lax.fori_loop(0, nzp, zfill, 0)
  plsc.subcore_barrier()

  def blk(b, carry):
    row0 = sid * RPW + b * J
    pltpu.sync_copy(dstr.at[pl.ds(row0, J)], idx_d)

    def jb(j, c2):
      pltpu.sync_copy(onesv, acc.at[idx_d.at[j]], add=True)
      return c2

    lax.fori_loop(0, J, jb, 0)
    return carry

  lax.fori_loop(0, NBLK, blk, 0)
  plsc.subcore_barrier()

  def wout(z, cr):
    pltpu.sync_copy(acc.at[pl.ds(base + z * zr, zr)],
                    out.at[pl.ds(base + z * zr, zr)])
    return cr

  lax.fori_loop(0, nzp, wout, 0)


def _sc_deg_body(zeros_h, ones_h, du_r, si_r, su_r, di_r,
                 deg_du, deg_si, deg_su, deg_di,
                 acc, idx_d, onesv):
  cid = lax.axis_index("c")
  sid = lax.axis_index("s")
  pltpu.sync_copy(ones_h, onesv)

  @pl.when(cid == 0)
  def _():
    _hist(du_r, deg_du, NUP, sid, acc, idx_d, onesv, zeros_h)
    _hist(si_r, deg_si, NIP, sid, acc, idx_d, onesv, zeros_h)

  @pl.when(cid == 1)
  def _():
    _hist(su_r, deg_su, NUP, sid, acc, idx_d, onesv, zeros_h)
    _hist(di_r, deg_di, NIP, sid, acc, idx_d, onesv, zeros_h)


def _sc_degrees(zeros_h, ones_h, du_r, si_r, su_r, di_r):
  mesh = plsc.VectorSubcoreMesh(core_axis_name="c", subcore_axis_name="s")
  return pl.kernel(
      _sc_deg_body,
      out_type=(jax.ShapeDtypeStruct((NUP, 16), jnp.float32),
                jax.ShapeDtypeStruct((NIP, 16), jnp.float32),
                jax.ShapeDtypeStruct((NUP, 16), jnp.float32),
                jax.ShapeDtypeStruct((NIP, 16), jnp.float32)),
      mesh=mesh,
      scratch_types=[
          pltpu.VMEM_SHARED((NUP, 16), jnp.float32),
          pltpu.VMEM((J, 128), jnp.int32),
          pltpu.VMEM((128, 16), jnp.float32),
      ],
      compiler_params=pltpu.CompilerParams(use_tc_tiling_on_sc=False),
  )(zeros_h, ones_h, du_r, si_r, su_r, di_r)


def _dis(deg_col):
  d = deg_col
  return jnp.where(d > 0, lax.rsqrt(jnp.maximum(d, 1.0)), 0.0)


def _row_mask(bu, nvalid):
  rid = pl.program_id(0) * bu + lax.broadcasted_iota(jnp.int32, (bu, 1), 0)
  return rid < nvalid


def _cat(agg):
  return jnp.concatenate([agg[j] for j in range(NCHUNK)], axis=-1)


def _split_store(o, val):
  for j in range(NCHUNK):
    o[j] = val[:, j * 16:(j + 1) * 16]


def _user_table0_body(x, w, b, deg, o):
  bu = o.shape[1]
  val = (jnp.dot(x[...], w[...], preferred_element_type=jnp.float32)
         + b[...]) * _dis(deg[:, 0:1])
  _split_store(o, jnp.where(_row_mask(bu, N_U), val, 0.0))


def _user_mid_body(agg, degd, degs, s, c2, o):
  bu = o.shape[1]
  h = jnp.maximum(_cat(agg) * _dis(degd[:, 0:1]) * s[...] + c2[...], 0.0)
  _split_store(o, jnp.where(_row_mask(bu, N_U), h * _dis(degs[:, 0:1]), 0.0))


def _user_fin_body(agg, degd, b, o):
  o[...] = _cat(agg) * _dis(degd[:, 0:1]) + b[...]


def _item_table0_body(x, w1, b1, w2, deg, o):
  bu = o.shape[1]
  h = jnp.dot(x[...], w1[...], preferred_element_type=jnp.float32) + b1[...]
  val = jnp.dot(h, w2[...], preferred_element_type=jnp.float32) \
      * _dis(deg[:, 0:1])
  _split_store(o, jnp.where(_row_mask(bu, N_I), val, 0.0))


def _item_mid_body(agg, degd, degs, w1, s, c2, w2, o):
  bu = o.shape[1]
  oi = jnp.dot(_cat(agg) * _dis(degd[:, 0:1]), w1[...],
               preferred_element_type=jnp.float32)
  h = jnp.maximum(oi * s[...] + c2[...], 0.0)
  val = jnp.dot(h, w2[...], preferred_element_type=jnp.float32) \
      * _dis(degs[:, 0:1])
  _split_store(o, jnp.where(_row_mask(bu, N_I), val, 0.0))


def _item_fin_body(agg, degd, w, b, o):
  oi = jnp.dot(_cat(agg) * _dis(degd[:, 0:1]), w[...],
               preferred_element_type=jnp.float32)
  o[...] = oi + b[...]


_BU = 512


def _rows_spec(width):
  return pl.BlockSpec((_BU, width), lambda i: (i, 0))


def _chunk_spec():
  return pl.BlockSpec((NCHUNK, _BU, 16), lambda i: (0, i, 0))


def _full_spec(shape):
  return pl.BlockSpec(shape, lambda i: tuple(0 for _ in shape))


def _tc_call(body, nrows, in_specs, args, chunked_out=True):
  if chunked_out:
    out_specs = _chunk_spec()
    out_shape = jax.ShapeDtypeStruct((NCHUNK, nrows, 16), jnp.float32)
  else:
    out_specs = _rows_spec(D)
    out_shape = jax.ShapeDtypeStruct((nrows, D), jnp.float32)
  return pl.pallas_call(
      body,
      grid=(nrows // _BU,),
      in_specs=in_specs,
      out_specs=out_specs,
      out_shape=out_shape,
  )(*args)


def _user_table0(x, w, b, deg):
  return _tc_call(
      _user_table0_body, NUP,
      [_rows_spec(D), _full_spec((D, D)), _full_spec((1, D)),
       _rows_spec(16)],
      (x, w, b.reshape(1, D), deg))


def _user_mid(agg, degd, degs, s, c2):
  return _tc_call(
      _user_mid_body, NUP,
      [_chunk_spec(), _rows_spec(16), _rows_spec(16), _full_spec((1, D)),
       _full_spec((1, D))],
      (agg, degd, degs, s.reshape(1, D), c2.reshape(1, D)))


def _user_fin(agg, degd, b):
  return _tc_call(
      _user_fin_body, NUP,
      [_chunk_spec(), _rows_spec(16), _full_spec((1, D))],
      (agg, degd, b.reshape(1, D)), chunked_out=False)


def _item_table0(x, w1, b1, w2, deg):
  return _tc_call(
      _item_table0_body, NIP,
      [_rows_spec(D), _full_spec((D, D)), _full_spec((1, D)),
       _full_spec((D, D)), _rows_spec(16)],
      (x, w1, b1.reshape(1, D), w2, deg))


def _item_mid(agg, degd, degs, w1, s, c2, w2):
  return _tc_call(
      _item_mid_body, NIP,
      [_chunk_spec(), _rows_spec(16), _rows_spec(16), _full_spec((D, D)),
       _full_spec((1, D)), _full_spec((1, D)), _full_spec((D, D))],
      (agg, degd, degs, w1, s.reshape(1, D), c2.reshape(1, D), w2))


def _item_fin(agg, degd, w, b):
  return _tc_call(
      _item_fin_body, NIP,
      [_chunk_spec(), _rows_spec(16), _full_spec((D, D)), _full_spec((1, D))],
      (agg, degd, w, b.reshape(1, D)), chunked_out=False)


def _pad_idx(idx, fill):
  p = jnp.concatenate(
      [idx, jnp.full((EPAD - E0,), fill, jnp.int32)])
  return p.reshape(NROWS, 128)


def kernel(x_user, x_item, edge_u2i, edge_i2u, lin_w_user, lin_b_user,
           lin_w_item, lin_b_item, w0_u2i, b0_u2i, w0_i2u, b0_i2u,
           w1_u2i, b1_u2i, w1_i2u, b1_i2u, w2_u2i, b2_u2i, w2_i2u, b2_i2u,
           g0_user, be0_user, g0_item, be0_item, g1_user, be1_user,
           g1_item, be1_item):
  su_r = _pad_idx(edge_u2i[0], N_U)
  di_r = _pad_idx(edge_u2i[1], N_I)
  si_r = _pad_idx(edge_i2u[0], N_I)
  du_r = _pad_idx(edge_i2u[1], N_U)

  zeros_h = jnp.zeros((ZRU, 16), jnp.float32)
  ones_h = jnp.ones((128, 16), jnp.float32)

  deg_du, deg_si, deg_su, deg_di = _sc_degrees(
      zeros_h, ones_h, du_r, si_r, su_r, di_r)

  inv = 1.0 / jnp.sqrt(jnp.float32(1.0 + EPS))
  s_u = (g0_user * inv, g1_user * inv)
  c2_u = (b0_i2u * s_u[0] + be0_user, b1_i2u * s_u[1] + be1_user)
  s_i = (g0_item * inv, g1_item * inv)
  c2_i = (b0_u2i * s_i[0] + be0_item, b1_u2i * s_i[1] + be1_item)
  w_u2i = (w0_u2i, w1_u2i, w2_u2i)
  w_i2u_next = (w1_i2u, w2_i2u)

  tu = _user_table0(x_user, lin_w_user, lin_b_user, deg_su)
  ti = _item_table0(x_item, lin_w_item, lin_b_item, w0_i2u, deg_si)

  for l in range(2):
    agg_u, agg_i = _sc_agg(zeros_h, tu, ti, su_r, di_r, si_r, du_r)
    tu = _user_mid(agg_u, deg_du, deg_su, s_u[l], c2_u[l])
    ti = _item_mid(agg_i, deg_di, deg_si, w_u2i[l], s_i[l], c2_i[l],
                   w_i2u_next[l])

  agg_u, agg_i = _sc_agg(zeros_h, tu, ti, su_r, di_r, si_r, du_r)
  out_u = _user_fin(agg_u, deg_du, b2_i2u)[:N_U]
  out_i = _item_fin(agg_i, deg_di, w2_u2i, b2_u2i)[:N_I]
  return (out_u, out_i)


# speedup vs baseline: 2.9753x; 1.1890x over previous
import jax
import jax.numpy as jnp
from jax import lax
from jax.experimental import pallas as pl
from jax.experimental.pallas import tpu as pltpu
from jax.experimental.pallas import tpu_sc as plsc

N_U = 100000
N_I = 10000
E0 = 500000
D = 128
EPS = 1e-5

EPAD = 524288
NROWS = EPAD // 128
RPW = NROWS // 16
J = 4
NBLK = RPW // J
ZRU = 196
ZRI = 160
NSB = NBLK // 2
NUP = 100352
NIP = 10240
NCHUNK = 8


def _run_dir(table, sdr, out, ndst, sid, acc, ibuf, gath,
             zeros_h, gsems, ssem, wsem):
  rows_w = ndst // 16
  base = sid * rows_w
  zr = ZRU if ndst == NUP else ZRI
  nzp = rows_w // zr
  for c in range(NCHUNK):
    tbl_c = table.at[c]

    def zfill(z, cr):
      pltpu.async_copy(zeros_h.at[pl.ds(0, zr)],
                       acc.at[pl.ds(base + z * zr, zr)], wsem)
      return cr

    def zdrain(z, cr):
      pltpu.make_async_copy(zeros_h.at[pl.ds(0, zr)],
                            acc.at[pl.ds(base + z * zr, zr)], wsem).wait()
      return cr

    lax.fori_loop(0, nzp, zfill, 0)
    lax.fori_loop(0, nzp, zdrain, 0)
    plsc.subcore_barrier()

    def load_idx(b, ph):
      row0 = (sid * NBLK + b) * 2 * J
      pltpu.sync_copy(sdr.at[pl.ds(row0, 2 * J)], ibuf[ph])

    def fire_gathers(ph):
      def f(j, cr):
        pltpu.async_copy(tbl_c.at[ibuf[ph].at[j]],
                         gath[ph].at[pl.ds(j * 128, 128)], gsems[ph])
        return cr
      lax.fori_loop(0, J, f, 0)

    def wait_gathers(ph):
      def f(j, cr):
        pltpu.make_async_copy(tbl_c.at[ibuf[ph].at[j]],
                              gath[ph].at[pl.ds(j * 128, 128)],
                              gsems[ph]).wait()
        return cr
      lax.fori_loop(0, J, f, 0)

    def fire_scatters(ph):
      def f(j, cr):
        pltpu.async_copy(gath[ph].at[pl.ds(j * 128, 128)],
                         acc.at[ibuf[ph].at[j + J]], ssem, add=True)
        return cr
      lax.fori_loop(0, J, f, 0)

    def drain_scatters(ph):
      def f(j, cr):
        pltpu.make_async_copy(zeros_h.at[pl.ds(0, 128)],
                              gath[ph].at[pl.ds(j * 128, 128)], ssem).wait()
        return cr
      lax.fori_loop(0, J, f, 0)

    load_idx(0, 0)
    fire_gathers(0)

    def sb_body(sb, carry):
      for ph in (0, 1):
        b = sb * 2 + ph
        if ph == 1:
          drain_scatters(0)
        else:
          @pl.when(sb > 0)
          def _():
            drain_scatters(1)
        if ph == 0:
          load_idx(b + 1, 1)
          wait_gathers(0)
          fire_gathers(1)
        else:
          @pl.when(sb < NSB - 1)
          def _():
            load_idx(b + 1, 0)
          wait_gathers(1)
          @pl.when(sb < NSB - 1)
          def _():
            fire_gathers(0)
        fire_scatters(ph)
      return carry

    lax.fori_loop(0, NSB, sb_body, 0)
    drain_scatters(1)
    plsc.subcore_barrier()
    out_c = out.at[c]

    def wout(z, cr):
      pltpu.async_copy(acc.at[pl.ds(base + z * zr, zr)],
                       out_c.at[pl.ds(base + z * zr, zr)], wsem)
      return cr

    def wdrain(z, cr):
      pltpu.make_async_copy(acc.at[pl.ds(base + z * zr, zr)],
                            out_c.at[pl.ds(base + z * zr, zr)], wsem).wait()
      return cr

    lax.fori_loop(0, nzp, wout, 0)
    lax.fori_loop(0, nzp, wdrain, 0)


def _sc_agg_body(zeros_h, tu, ti, sd_u2i, sd_i2u, agg_u, agg_i,
                 acc, ib0, ib1, g0, g1, gsem0, gsem1, ssem, wsem):
  cid = lax.axis_index("c")
  sid = lax.axis_index("s")
  ibuf = (ib0, ib1)
  gath = (g0, g1)
  gsems = (gsem0, gsem1)

  @pl.when(cid == 0)
  def _():
    _run_dir(ti, sd_i2u, agg_u, NUP, sid, acc, ibuf, gath,
             zeros_h, gsems, ssem, wsem)

  @pl.when(cid == 1)
  def _():
    _run_dir(tu, sd_u2i, agg_i, NIP, sid, acc, ibuf, gath,
             zeros_h, gsems, ssem, wsem)


def _sc_agg(zeros_h, tu, ti, sd_u2i, sd_i2u):
  mesh = plsc.VectorSubcoreMesh(core_axis_name="c", subcore_axis_name="s")
  return pl.kernel(
      _sc_agg_body,
      out_type=(jax.ShapeDtypeStruct((NCHUNK, NUP, 16), jnp.float32),
                jax.ShapeDtypeStruct((NCHUNK, NIP, 16), jnp.float32)),
      mesh=mesh,
      scratch_types=[
          pltpu.VMEM_SHARED((NUP, 16), jnp.float32),
          pltpu.VMEM((2 * J, 128), jnp.int32),
          pltpu.VMEM((2 * J, 128), jnp.int32),
          pltpu.VMEM((J * 128, 16), jnp.float32),
          pltpu.VMEM((J * 128, 16), jnp.float32),
          pltpu.SemaphoreType.DMA,
          pltpu.SemaphoreType.DMA,
          pltpu.SemaphoreType.DMA,
          pltpu.SemaphoreType.DMA,
      ],
      compiler_params=pltpu.CompilerParams(use_tc_tiling_on_sc=False),
  )(zeros_h, tu, ti, sd_u2i, sd_i2u)


def _hist(dstr, out, ndst, sid, acc, idx_d, onesv, zeros_h):
  rows_w = ndst // 16
  base = sid * rows_w
  zr = ZRU if ndst == NUP else ZRI
  nzp = rows_w // zr

  def zfill(z, cr):
    pltpu.sync_copy(zeros_h.at[pl.ds(0, zr)],
                    acc.at[pl.ds(base + z * zr, zr)])
    return cr

  lax.fori_loop(0, nzp, zfill, 0)
  plsc.subcore_barrier()

  def blk(b, carry):
    row0 = sid * RPW + b * J
    pltpu.sync_copy(dstr.at[pl.ds(row0, J)], idx_d)

    def jb(j, c2):
      pltpu.sync_copy(onesv, acc.at[idx_d.at[j]], add=True)
      return c2

    lax.fori_loop(0, J, jb, 0)
    return carry

  lax.fori_loop(0, NBLK, blk, 0)
  plsc.subcore_barrier()

  def wout(z, cr):
    pltpu.sync_copy(acc.at[pl.ds(base + z * zr, zr)],
                    out.at[pl.ds(base + z * zr, zr)])
    return cr

  lax.fori_loop(0, nzp, wout, 0)


def _sc_deg_body(zeros_h, ones_h, du_r, si_r, su_r, di_r,
                 deg_du, deg_si, deg_su, deg_di,
                 acc, idx_d, onesv):
  cid = lax.axis_index("c")
  sid = lax.axis_index("s")
  pltpu.sync_copy(ones_h, onesv)

  @pl.when(cid == 0)
  def _():
    _hist(du_r, deg_du, NUP, sid, acc, idx_d, onesv, zeros_h)
    _hist(si_r, deg_si, NIP, sid, acc, idx_d, onesv, zeros_h)

  @pl.when(cid == 1)
  def _():
    _hist(su_r, deg_su, NUP, sid, acc, idx_d, onesv, zeros_h)
    _hist(di_r, deg_di, NIP, sid, acc, idx_d, onesv, zeros_h)


def _sc_degrees(zeros_h, ones_h, du_r, si_r, su_r, di_r):
  mesh = plsc.VectorSubcoreMesh(core_axis_name="c", subcore_axis_name="s")
  return pl.kernel(
      _sc_deg_body,
      out_type=(jax.ShapeDtypeStruct((NUP, 16), jnp.float32),
                jax.ShapeDtypeStruct((NIP, 16), jnp.float32),
                jax.ShapeDtypeStruct((NUP, 16), jnp.float32),
                jax.ShapeDtypeStruct((NIP, 16), jnp.float32)),
      mesh=mesh,
      scratch_types=[
          pltpu.VMEM_SHARED((NUP, 16), jnp.float32),
          pltpu.VMEM((J, 128), jnp.int32),
          pltpu.VMEM((128, 16), jnp.float32),
      ],
      compiler_params=pltpu.CompilerParams(use_tc_tiling_on_sc=False),
  )(zeros_h, ones_h, du_r, si_r, su_r, di_r)


def _dis(deg_col):
  d = deg_col
  return jnp.where(d > 0, lax.rsqrt(jnp.maximum(d, 1.0)), 0.0)


def _row_mask(bu, nvalid):
  rid = pl.program_id(0) * bu + lax.broadcasted_iota(jnp.int32, (bu, 1), 0)
  return rid < nvalid


def _cat(agg):
  return jnp.concatenate([agg[j] for j in range(NCHUNK)], axis=-1)


def _split_store(o, val):
  for j in range(NCHUNK):
    o[j] = val[:, j * 16:(j + 1) * 16]


def _user_table0_body(x, w, b, deg, o):
  bu = o.shape[1]
  val = (jnp.dot(x[...], w[...], preferred_element_type=jnp.float32)
         + b[...]) * _dis(deg[:, 0:1])
  _split_store(o, jnp.where(_row_mask(bu, N_U), val, 0.0))


def _user_mid_body(agg, degd, degs, s, c2, o):
  bu = o.shape[1]
  h = jnp.maximum(_cat(agg) * _dis(degd[:, 0:1]) * s[...] + c2[...], 0.0)
  _split_store(o, jnp.where(_row_mask(bu, N_U), h * _dis(degs[:, 0:1]), 0.0))


def _user_fin_body(agg, degd, b, o):
  o[...] = _cat(agg) * _dis(degd[:, 0:1]) + b[...]


def _item_table0_body(x, w1, b1, w2, deg, o):
  bu = o.shape[1]
  h = jnp.dot(x[...], w1[...], preferred_element_type=jnp.float32) + b1[...]
  val = jnp.dot(h, w2[...], preferred_element_type=jnp.float32) \
      * _dis(deg[:, 0:1])
  _split_store(o, jnp.where(_row_mask(bu, N_I), val, 0.0))


def _item_mid_body(agg, degd, degs, w1, s, c2, w2, o):
  bu = o.shape[1]
  oi = jnp.dot(_cat(agg) * _dis(degd[:, 0:1]), w1[...],
               preferred_element_type=jnp.float32)
  h = jnp.maximum(oi * s[...] + c2[...], 0.0)
  val = jnp.dot(h, w2[...], preferred_element_type=jnp.float32) \
      * _dis(degs[:, 0:1])
  _split_store(o, jnp.where(_row_mask(bu, N_I), val, 0.0))


def _item_fin_body(agg, degd, w, b, o):
  oi = jnp.dot(_cat(agg) * _dis(degd[:, 0:1]), w[...],
               preferred_element_type=jnp.float32)
  o[...] = oi + b[...]


_BU = 512


def _rows_spec(width):
  return pl.BlockSpec((_BU, width), lambda i: (i, 0))


def _chunk_spec():
  return pl.BlockSpec((NCHUNK, _BU, 16), lambda i: (0, i, 0))


def _full_spec(shape):
  return pl.BlockSpec(shape, lambda i: tuple(0 for _ in shape))


def _tc_call(body, nrows, in_specs, args, chunked_out=True):
  if chunked_out:
    out_specs = _chunk_spec()
    out_shape = jax.ShapeDtypeStruct((NCHUNK, nrows, 16), jnp.float32)
  else:
    out_specs = _rows_spec(D)
    out_shape = jax.ShapeDtypeStruct((nrows, D), jnp.float32)
  return pl.pallas_call(
      body,
      grid=(nrows // _BU,),
      in_specs=in_specs,
      out_specs=out_specs,
      out_shape=out_shape,
  )(*args)


def _user_table0(x, w, b, deg):
  return _tc_call(
      _user_table0_body, NUP,
      [_rows_spec(D), _full_spec((D, D)), _full_spec((1, D)),
       _rows_spec(16)],
      (x, w, b.reshape(1, D), deg))


def _user_mid(agg, degd, degs, s, c2):
  return _tc_call(
      _user_mid_body, NUP,
      [_chunk_spec(), _rows_spec(16), _rows_spec(16), _full_spec((1, D)),
       _full_spec((1, D))],
      (agg, degd, degs, s.reshape(1, D), c2.reshape(1, D)))


def _user_fin(agg, degd, b):
  return _tc_call(
      _user_fin_body, NUP,
      [_chunk_spec(), _rows_spec(16), _full_spec((1, D))],
      (agg, degd, b.reshape(1, D)), chunked_out=False)


def _item_table0(x, w1, b1, w2, deg):
  return _tc_call(
      _item_table0_body, NIP,
      [_rows_spec(D), _full_spec((D, D)), _full_spec((1, D)),
       _full_spec((D, D)), _rows_spec(16)],
      (x, w1, b1.reshape(1, D), w2, deg))


def _item_mid(agg, degd, degs, w1, s, c2, w2):
  return _tc_call(
      _item_mid_body, NIP,
      [_chunk_spec(), _rows_spec(16), _rows_spec(16), _full_spec((D, D)),
       _full_spec((1, D)), _full_spec((1, D)), _full_spec((D, D))],
      (agg, degd, degs, w1, s.reshape(1, D), c2.reshape(1, D), w2))


def _item_fin(agg, degd, w, b):
  return _tc_call(
      _item_fin_body, NIP,
      [_chunk_spec(), _rows_spec(16), _full_spec((D, D)), _full_spec((1, D))],
      (agg, degd, w, b.reshape(1, D)), chunked_out=False)


def _pad_idx(idx, fill):
  p = jnp.concatenate(
      [idx, jnp.full((EPAD - E0,), fill, jnp.int32)])
  return p.reshape(NROWS, 128)


def _interleave(src_r, dst_r):
  a = src_r.reshape(NROWS // J, J, 128)
  b = dst_r.reshape(NROWS // J, J, 128)
  return jnp.concatenate([a, b], axis=1).reshape(2 * NROWS, 128)


def kernel(x_user, x_item, edge_u2i, edge_i2u, lin_w_user, lin_b_user,
           lin_w_item, lin_b_item, w0_u2i, b0_u2i, w0_i2u, b0_i2u,
           w1_u2i, b1_u2i, w1_i2u, b1_i2u, w2_u2i, b2_u2i, w2_i2u, b2_i2u,
           g0_user, be0_user, g0_item, be0_item, g1_user, be1_user,
           g1_item, be1_item):
  su_r = _pad_idx(edge_u2i[0], N_U)
  di_r = _pad_idx(edge_u2i[1], N_I)
  si_r = _pad_idx(edge_i2u[0], N_I)
  du_r = _pad_idx(edge_i2u[1], N_U)
  sd_u2i = _interleave(su_r, di_r)
  sd_i2u = _interleave(si_r, du_r)

  zeros_h = jnp.zeros((ZRU, 16), jnp.float32)
  ones_h = jnp.ones((128, 16), jnp.float32)

  deg_du, deg_si, deg_su, deg_di = _sc_degrees(
      zeros_h, ones_h, du_r, si_r, su_r, di_r)

  inv = 1.0 / jnp.sqrt(jnp.float32(1.0 + EPS))
  s_u = (g0_user * inv, g1_user * inv)
  c2_u = (b0_i2u * s_u[0] + be0_user, b1_i2u * s_u[1] + be1_user)
  s_i = (g0_item * inv, g1_item * inv)
  c2_i = (b0_u2i * s_i[0] + be0_item, b1_u2i * s_i[1] + be1_item)
  w_u2i = (w0_u2i, w1_u2i, w2_u2i)
  w_i2u_next = (w1_i2u, w2_i2u)

  tu = _user_table0(x_user, lin_w_user, lin_b_user, deg_su)
  ti = _item_table0(x_item, lin_w_item, lin_b_item, w0_i2u, deg_si)

  for l in range(2):
    agg_u, agg_i = _sc_agg(zeros_h, tu, ti, sd_u2i, sd_i2u)
    tu = _user_mid(agg_u, deg_du, deg_su, s_u[l], c2_u[l])
    ti = _item_mid(agg_i, deg_di, deg_si, w_u2i[l], s_i[l], c2_i[l],
                   w_i2u_next[l])

  agg_u, agg_i = _sc_agg(zeros_h, tu, ti, sd_u2i, sd_i2u)
  out_u = _user_fin(agg_u, deg_du, b2_i2u)[:N_U]
  out_i = _item_fin(agg_i, deg_di, w2_u2i, b2_u2i)[:N_I]
  return (out_u, out_i)
